# Initial kernel scaffold; baseline (speedup 1.0000x reference)
import jax
import jax.numpy as jnp
from jax.experimental import pallas as pl


def kernel(x, edge_index, edge_feat, W1, b1, W2, b2, We, be, Wmu, bmu, Wlv, blv):
    raise NotImplementedError("write your pallas kernel here")


# speedup vs baseline: 10.8359x; 10.8359x over previous
import functools

import jax
import jax.numpy as jnp
from jax import lax
from jax.experimental import pallas as pl
from jax.experimental.pallas import tpu as pltpu
from jax.experimental.pallas import tpu_sc as plsc

N = 10000
E = 320000
IN_DIM = 128
HID = 64
LAT = 16

NC = 2
NS = 16
NW = NC * NS
NPAD = 10240
SLICE = NPAD // NS
EPW = E // NW
WIN = 80
NWIN = EPW // WIN

_mesh = plsc.VectorSubcoreMesh(core_axis_name="c", subcore_axis_name="s")
_sc_params = pltpu.CompilerParams(use_tc_tiling_on_sc=False)


@functools.partial(
    pl.kernel,
    out_type=(
        jax.ShapeDtypeStruct((NC, NPAD), jnp.float32),
        jax.ShapeDtypeStruct((NC, NPAD), jnp.float32),
    ),
    mesh=_mesh,
    compiler_params=_sc_params,
    scratch_types=[
        pltpu.VMEM((NWIN, WIN), jnp.int32),
        pltpu.VMEM((WIN,), jnp.float32),
        pltpu.VMEM((SLICE,), jnp.float32),
        pltpu.VMEM_SHARED((NPAD,), jnp.float32),
        pltpu.VMEM_SHARED((NPAD,), jnp.float32),
    ],
)
def _deg_kernel(src_hbm, dst_hbm, dout_hbm, din_hbm,
                idx_v, ones_v, zero_v, dout_sp, din_sp):
    c = lax.axis_index("c")
    s = lax.axis_index("s")
    w = c * NS + s
    for j in range(SLICE // 16):
        zero_v[pl.ds(16 * j, 16)] = jnp.zeros((16,), jnp.float32)
    for j in range(WIN // 16):
        ones_v[pl.ds(16 * j, 16)] = jnp.ones((16,), jnp.float32)
    pltpu.sync_copy(zero_v, dout_sp.at[pl.ds(s * SLICE, SLICE)])
    pltpu.sync_copy(zero_v, din_sp.at[pl.ds(s * SLICE, SLICE)])
    plsc.subcore_barrier()
    pltpu.sync_copy(src_hbm.at[w], idx_v)

    def body_out(j, carry):
        pltpu.sync_copy(ones_v, dout_sp.at[idx_v.at[j]], add=True)
        return carry

    lax.fori_loop(0, NWIN, body_out, 0)
    pltpu.sync_copy(dst_hbm.at[w], idx_v)

    def body_in(j, carry):
        pltpu.sync_copy(ones_v, din_sp.at[idx_v.at[j]], add=True)
        return carry

    lax.fori_loop(0, NWIN, body_in, 0)
    plsc.subcore_barrier()
    pltpu.sync_copy(dout_sp.at[pl.ds(s * SLICE, SLICE)],
                    dout_hbm.at[c, pl.ds(s * SLICE, SLICE)])
    pltpu.sync_copy(din_sp.at[pl.ds(s * SLICE, SLICE)],
                    din_hbm.at[c, pl.ds(s * SLICE, SLICE)])


@functools.partial(
    pl.kernel,
    out_type=jax.ShapeDtypeStruct((NC, NPAD, HID), jnp.float32),
    mesh=_mesh,
    compiler_params=_sc_params,
    scratch_types=[
        pltpu.VMEM((NWIN, WIN), jnp.int32),
        pltpu.VMEM((NWIN, WIN), jnp.int32),
        pltpu.VMEM((WIN, HID), jnp.float32),
        pltpu.VMEM_SHARED((NPAD, HID), jnp.float32),
        pltpu.VMEM_SHARED((NPAD, HID), jnp.float32),
    ],
)
def _scatter_kernel(t_hbm, src_hbm, dst_hbm, zeros_hbm, out_hbm,
                    src_v, dst_v, rows_v, t_sp, acc_sp):
    c = lax.axis_index("c")
    s = lax.axis_index("s")
    w = c * NS + s
    pltpu.sync_copy(t_hbm.at[pl.ds(s * SLICE, SLICE)],
                    t_sp.at[pl.ds(s * SLICE, SLICE)])
    pltpu.sync_copy(zeros_hbm.at[pl.ds(s * SLICE, SLICE)],
                    acc_sp.at[pl.ds(s * SLICE, SLICE)])
    pltpu.sync_copy(src_hbm.at[w], src_v)
    pltpu.sync_copy(dst_hbm.at[w], dst_v)
    plsc.subcore_barrier()

    def body(j, carry):
        pltpu.sync_copy(t_sp.at[src_v.at[j]], rows_v)
        pltpu.sync_copy(rows_v, acc_sp.at[dst_v.at[j]], add=True)
        return carry

    lax.fori_loop(0, NWIN, body, 0)
    plsc.subcore_barrier()
    pltpu.sync_copy(acc_sp.at[pl.ds(s * SLICE, SLICE)],
                    out_hbm.at[c, pl.ds(s * SLICE, SLICE)])


def _tc_proj1(xpad, W1, dout_col):
    def body(x_ref, w_ref, dg_ref, o_ref):
        scale = lax.rsqrt(jnp.maximum(dg_ref[...], 1.0))
        o_ref[...] = jnp.dot(x_ref[...], w_ref[...],
                             preferred_element_type=jnp.float32) * scale

    return pl.pallas_call(
        body,
        out_shape=jax.ShapeDtypeStruct((NPAD, HID), jnp.float32),
    )(xpad, W1, dout_col)


def _tc_mid(acc, din_col, dout_col, b1_row, W2):
    def body(a_ref, din_ref, dout_ref, b_ref, w_ref, o_ref):
        din_s = lax.rsqrt(jnp.maximum(din_ref[...], 1.0))
        dout_s = lax.rsqrt(jnp.maximum(dout_ref[...], 1.0))
        h = jnp.maximum((a_ref[0] + a_ref[1]) * din_s + b_ref[...], 0.0)
        o_ref[...] = jnp.dot(h, w_ref[...],
                             preferred_element_type=jnp.float32) * dout_s

    return pl.pallas_call(
        body,
        out_shape=jax.ShapeDtypeStruct((NPAD, HID), jnp.float32),
    )(acc, din_col, dout_col, b1_row, W2)


def _tc_head(acc, din_col, b2_row, Wmu, bmu_row, Wlv, blv_row):
    def body(a_ref, din_ref, b_ref, wmu_ref, bmu_ref, wlv_ref, blv_ref, o_ref):
        din_s = lax.rsqrt(jnp.maximum(din_ref[...], 1.0))
        h = jnp.maximum((a_ref[0] + a_ref[1]) * din_s + b_ref[...], 0.0)
        hg = jnp.sum(h[:N], axis=0, keepdims=True) * (1.0 / N)
        mu = jnp.dot(hg, wmu_ref[...], preferred_element_type=jnp.float32)
        lv = jnp.dot(hg, wlv_ref[...], preferred_element_type=jnp.float32)
        o_ref[...] = jnp.concatenate([mu + bmu_ref[...], lv + blv_ref[...]],
                                     axis=0)

    return pl.pallas_call(
        body,
        out_shape=jax.ShapeDtypeStruct((2, LAT), jnp.float32),
    )(acc, din_col, b2_row, Wmu, bmu_row, Wlv, blv_row)


def kernel(x, edge_index, edge_feat, W1, b1, W2, b2, We, be, Wmu, bmu, Wlv, blv):
    src3 = edge_index[0].reshape(NW, NWIN, WIN)
    dst3 = edge_index[1].reshape(NW, NWIN, WIN)
    zeros2d = jnp.zeros((NPAD, HID), jnp.float32)
    xpad = jnp.concatenate(
        [x, jnp.zeros((NPAD - N, IN_DIM), jnp.float32)], axis=0)

    deg_out2, deg_in2 = _deg_kernel(src3, dst3)
    dout_col = (deg_out2[0] + deg_out2[1]).reshape(NPAD, 1)
    din_col = (deg_in2[0] + deg_in2[1]).reshape(NPAD, 1)

    t1 = _tc_proj1(xpad, W1, dout_col)
    acc1 = _scatter_kernel(t1, src3, dst3, zeros2d)
    t2 = _tc_mid(acc1, din_col, dout_col, b1.reshape(1, HID), W2)
    acc2 = _scatter_kernel(t2, src3, dst3, zeros2d)
    out = _tc_head(acc2, din_col, b2.reshape(1, HID),
                   Wmu, bmu.reshape(1, LAT), Wlv, blv.reshape(1, LAT))
    return out[0:1], out[1:2]


# speedup vs baseline: 14.2277x; 1.3130x over previous
import functools

import jax
import jax.numpy as jnp
from jax import lax
from jax.experimental import pallas as pl
from jax.experimental.pallas import tpu as pltpu
from jax.experimental.pallas import tpu_sc as plsc

N = 10000
E = 320000
IN_DIM = 128
HID = 64
LAT = 16

NC = 2
NS = 16
NW = NC * NS
NPAD = 10240
SLICE = NPAD // NS
WIN = 128
EPW = E // NW
EPW_PAD = 10240
NWIN = EPW_PAD // WIN
PIPE = 8
NBLK = NWIN // PIPE

_mesh = plsc.VectorSubcoreMesh(core_axis_name="c", subcore_axis_name="s")
_sc_params = pltpu.CompilerParams(use_tc_tiling_on_sc=False)


@functools.partial(
    pl.kernel,
    out_type=(
        jax.ShapeDtypeStruct((NC, NPAD), jnp.float32),
        jax.ShapeDtypeStruct((NC, NPAD), jnp.float32),
    ),
    mesh=_mesh,
    compiler_params=_sc_params,
    scratch_types=[
        pltpu.VMEM((NWIN, WIN), jnp.int32),
        pltpu.VMEM((NWIN, WIN), jnp.int32),
        pltpu.VMEM((WIN,), jnp.float32),
        pltpu.VMEM((SLICE,), jnp.float32),
        pltpu.VMEM_SHARED((NPAD,), jnp.float32),
        pltpu.VMEM_SHARED((NPAD,), jnp.float32),
    ]
    + [pltpu.SemaphoreType.DMA] * (2 * PIPE),
)
def _deg_kernel(src_hbm, dst_hbm, dout_hbm, din_hbm,
                src_v, dst_v, ones_v, zero_v, dout_sp, din_sp, *sems):
    c = lax.axis_index("c")
    s = lax.axis_index("s")
    w = c * NS + s
    for j in range(SLICE // 16):
        zero_v[pl.ds(16 * j, 16)] = jnp.zeros((16,), jnp.float32)
    for j in range(WIN // 16):
        ones_v[pl.ds(16 * j, 16)] = jnp.ones((16,), jnp.float32)
    pltpu.sync_copy(zero_v, dout_sp.at[pl.ds(s * SLICE, SLICE)])
    pltpu.sync_copy(zero_v, din_sp.at[pl.ds(s * SLICE, SLICE)])
    pltpu.sync_copy(src_hbm.at[w], src_v)
    pltpu.sync_copy(dst_hbm.at[w], dst_v)
    plsc.subcore_barrier()

    def body(k, carry):
        ds = []
        for b in range(PIPE):
            j = k * PIPE + b
            ds.append(pltpu.async_copy(
                ones_v, dout_sp.at[src_v.at[j]], sems[b], add=True))
            ds.append(pltpu.async_copy(
                ones_v, din_sp.at[dst_v.at[j]], sems[PIPE + b], add=True))
        for d in ds:
            d.wait()
        return carry

    lax.fori_loop(0, NBLK, body, 0)
    plsc.subcore_barrier()
    pltpu.sync_copy(dout_sp.at[pl.ds(s * SLICE, SLICE)],
                    dout_hbm.at[c, pl.ds(s * SLICE, SLICE)])
    pltpu.sync_copy(din_sp.at[pl.ds(s * SLICE, SLICE)],
                    din_hbm.at[c, pl.ds(s * SLICE, SLICE)])


@functools.partial(
    pl.kernel,
    out_type=jax.ShapeDtypeStruct((NC, NPAD, HID), jnp.float32),
    mesh=_mesh,
    compiler_params=_sc_params,
    scratch_types=[
        pltpu.VMEM((NWIN, WIN), jnp.int32),
        pltpu.VMEM((NWIN, WIN), jnp.int32),
        pltpu.VMEM_SHARED((NPAD, HID), jnp.float32),
    ]
    + [pltpu.VMEM((WIN, HID), jnp.float32)] * PIPE
    + [pltpu.SemaphoreType.DMA] * (2 * PIPE),
)
def _scatter_kernel(t_hbm, src_hbm, dst_hbm, zeros_hbm, out_hbm,
                    src_v, dst_v, acc_sp, *bufs_and_sems):
    bufs = bufs_and_sems[:PIPE]
    gsem = bufs_and_sems[PIPE:2 * PIPE]
    ssem = bufs_and_sems[2 * PIPE:]
    c = lax.axis_index("c")
    s = lax.axis_index("s")
    w = c * NS + s
    pltpu.sync_copy(zeros_hbm.at[pl.ds(s * SLICE, SLICE)],
                    acc_sp.at[pl.ds(s * SLICE, SLICE)])
    pltpu.sync_copy(src_hbm.at[w], src_v)
    pltpu.sync_copy(dst_hbm.at[w], dst_v)
    plsc.subcore_barrier()

    def body(k, carry):
        gs = []
        for b in range(PIPE):
            j = k * PIPE + b
            gs.append(pltpu.async_copy(
                t_hbm.at[src_v.at[j]], bufs[b], gsem[b]))
        ss = []
        for b in range(PIPE):
            j = k * PIPE + b
            gs[b].wait()
            ss.append(pltpu.async_copy(
                bufs[b], acc_sp.at[dst_v.at[j]], ssem[b], add=True))
        for d in ss:
            d.wait()
        return carry

    lax.fori_loop(0, NBLK, body, 0)
    plsc.subcore_barrier()
    pltpu.sync_copy(acc_sp.at[pl.ds(s * SLICE, SLICE)],
                    out_hbm.at[c, pl.ds(s * SLICE, SLICE)])


def _tc_proj1(xpad, W1, dout_col):
    def body(x_ref, w_ref, dg_ref, o_ref):
        scale = lax.rsqrt(jnp.maximum(dg_ref[...], 1.0))
        o_ref[...] = jnp.dot(x_ref[...], w_ref[...],
                             preferred_element_type=jnp.float32) * scale

    return pl.pallas_call(
        body,
        out_shape=jax.ShapeDtypeStruct((NPAD, HID), jnp.float32),
    )(xpad, W1, dout_col)


def _tc_mid(acc, din_col, dout_col, b1_row, W2):
    def body(a_ref, din_ref, dout_ref, b_ref, w_ref, o_ref):
        din_s = lax.rsqrt(jnp.maximum(din_ref[...], 1.0))
        dout_s = lax.rsqrt(jnp.maximum(dout_ref[...], 1.0))
        h = jnp.maximum((a_ref[0] + a_ref[1]) * din_s + b_ref[...], 0.0)
        row = lax.broadcasted_iota(jnp.int32, (NPAD, 1), 0)
        t2 = jnp.dot(h, w_ref[...],
                     preferred_element_type=jnp.float32) * dout_s
        o_ref[...] = jnp.where(row < N, t2, 0.0)

    return pl.pallas_call(
        body,
        out_shape=jax.ShapeDtypeStruct((NPAD, HID), jnp.float32),
    )(acc, din_col, dout_col, b1_row, W2)


def _tc_head(acc, din_col, b2_row, Wmu, bmu_row, Wlv, blv_row):
    def body(a_ref, din_ref, b_ref, wmu_ref, bmu_ref, wlv_ref, blv_ref, o_ref):
        din_s = lax.rsqrt(jnp.maximum(din_ref[...], 1.0))
        h = jnp.maximum((a_ref[0] + a_ref[1]) * din_s + b_ref[...], 0.0)
        hg = jnp.sum(h[:N], axis=0, keepdims=True) * (1.0 / N)
        mu = jnp.dot(hg, wmu_ref[...], preferred_element_type=jnp.float32)
        lv = jnp.dot(hg, wlv_ref[...], preferred_element_type=jnp.float32)
        o_ref[...] = jnp.concatenate([mu + bmu_ref[...], lv + blv_ref[...]],
                                     axis=0)

    return pl.pallas_call(
        body,
        out_shape=jax.ShapeDtypeStruct((2, LAT), jnp.float32),
    )(acc, din_col, b2_row, Wmu, bmu_row, Wlv, blv_row)


def kernel(x, edge_index, edge_feat, W1, b1, W2, b2, We, be, Wmu, bmu, Wlv, blv):
    pad_idx = N + (jnp.arange(EPW_PAD - EPW, dtype=jnp.int32) % (NPAD - N))
    pad_blk = jnp.broadcast_to(pad_idx, (NW, EPW_PAD - EPW))
    src3 = jnp.concatenate(
        [edge_index[0].reshape(NW, EPW), pad_blk], axis=1).reshape(NW, NWIN, WIN)
    dst3 = jnp.concatenate(
        [edge_index[1].reshape(NW, EPW), pad_blk], axis=1).reshape(NW, NWIN, WIN)
    zeros2d = jnp.zeros((NPAD, HID), jnp.float32)
    xpad = jnp.concatenate(
        [x, jnp.zeros((NPAD - N, IN_DIM), jnp.float32)], axis=0)

    deg_out2, deg_in2 = _deg_kernel(src3, dst3)
    dout_col = (deg_out2[0] + deg_out2[1]).reshape(NPAD, 1)
    din_col = (deg_in2[0] + deg_in2[1]).reshape(NPAD, 1)

    t1 = _tc_proj1(xpad, W1, dout_col)
    acc1 = _scatter_kernel(t1, src3, dst3, zeros2d)
    t2 = _tc_mid(acc1, din_col, dout_col, b1.reshape(1, HID), W2)
    acc2 = _scatter_kernel(t2, src3, dst3, zeros2d)
    out = _tc_head(acc2, din_col, b2.reshape(1, HID),
                   Wmu, bmu.reshape(1, LAT), Wlv, blv.reshape(1, LAT))
    return out[0:1], out[1:2]


# speedup vs baseline: 16.6205x; 1.1682x over previous
import functools

import jax
import jax.numpy as jnp
from jax import lax
from jax.experimental import pallas as pl
from jax.experimental.pallas import tpu as pltpu
from jax.experimental.pallas import tpu_sc as plsc

N = 10000
E = 320000
IN_DIM = 128
HID = 64
LAT = 16

NC = 2
NS = 16
NW = NC * NS
NPAD = 10240
SLICE = NPAD // NS
WIN = 128
EPW = E // NW
EPW_PAD = 10240
NWIN = EPW_PAD // WIN
PIPE = 8
NBLK = NWIN // PIPE

_mesh = plsc.VectorSubcoreMesh(core_axis_name="c", subcore_axis_name="s")
_sc_params = pltpu.CompilerParams(use_tc_tiling_on_sc=False)


@functools.partial(
    pl.kernel,
    out_type=(
        jax.ShapeDtypeStruct((NC, NPAD), jnp.float32),
        jax.ShapeDtypeStruct((NC, NPAD), jnp.float32),
    ),
    mesh=_mesh,
    compiler_params=_sc_params,
    scratch_types=[
        pltpu.VMEM((NWIN, WIN), jnp.int32),
        pltpu.VMEM((NWIN, WIN), jnp.int32),
        pltpu.VMEM((WIN,), jnp.float32),
        pltpu.VMEM((SLICE,), jnp.float32),
        pltpu.VMEM_SHARED((NPAD,), jnp.float32),
        pltpu.VMEM_SHARED((NPAD,), jnp.float32),
    ]
    + [pltpu.SemaphoreType.DMA] * (2 * PIPE),
)
def _deg_kernel(src_hbm, dst_hbm, dout_hbm, din_hbm,
                src_v, dst_v, ones_v, zero_v, dout_sp, din_sp, *sems):
    c = lax.axis_index("c")
    s = lax.axis_index("s")
    w = c * NS + s
    for j in range(SLICE // 16):
        zero_v[pl.ds(16 * j, 16)] = jnp.zeros((16,), jnp.float32)
    for j in range(WIN // 16):
        ones_v[pl.ds(16 * j, 16)] = jnp.ones((16,), jnp.float32)
    pltpu.sync_copy(zero_v, dout_sp.at[pl.ds(s * SLICE, SLICE)])
    pltpu.sync_copy(zero_v, din_sp.at[pl.ds(s * SLICE, SLICE)])
    pltpu.sync_copy(src_hbm.at[w], src_v)
    pltpu.sync_copy(dst_hbm.at[w], dst_v)
    plsc.subcore_barrier()

    def body(k, carry):
        ds = []
        for b in range(PIPE):
            j = k * PIPE + b
            ds.append(pltpu.async_copy(
                ones_v, dout_sp.at[src_v.at[j]], sems[b], add=True))
            ds.append(pltpu.async_copy(
                ones_v, din_sp.at[dst_v.at[j]], sems[PIPE + b], add=True))
        for d in ds:
            d.wait()
        return carry

    lax.fori_loop(0, NBLK, body, 0)
    plsc.subcore_barrier()
    pltpu.sync_copy(dout_sp.at[pl.ds(s * SLICE, SLICE)],
                    dout_hbm.at[c, pl.ds(s * SLICE, SLICE)])
    pltpu.sync_copy(din_sp.at[pl.ds(s * SLICE, SLICE)],
                    din_hbm.at[c, pl.ds(s * SLICE, SLICE)])


@functools.partial(
    pl.kernel,
    out_type=jax.ShapeDtypeStruct((NC, NPAD, HID), jnp.float32),
    mesh=_mesh,
    compiler_params=_sc_params,
    scratch_types=[
        pltpu.VMEM((NWIN, WIN), jnp.int32),
        pltpu.VMEM((NWIN, WIN), jnp.int32),
        pltpu.VMEM_SHARED((NPAD, HID), jnp.float32),
    ]
    + [pltpu.VMEM((WIN, HID), jnp.float32)] * PIPE
    + [pltpu.SemaphoreType.DMA] * PIPE,
)
def _scatter_kernel(t_hbm, src_hbm, dst_hbm, zeros_hbm, out_hbm,
                    src_v, dst_v, acc_sp, *bufs_and_sems):
    bufs = bufs_and_sems[:PIPE]
    gsem = bufs_and_sems[PIPE:]
    c = lax.axis_index("c")
    s = lax.axis_index("s")
    w = c * NS + s
    pltpu.sync_copy(zeros_hbm.at[pl.ds(s * SLICE, SLICE)],
                    acc_sp.at[pl.ds(s * SLICE, SLICE)])
    pltpu.sync_copy(src_hbm.at[w], src_v)
    pltpu.sync_copy(dst_hbm.at[w], dst_v)
    plsc.subcore_barrier()

    for b in range(PIPE):
        pltpu.async_copy(t_hbm.at[src_v.at[b]], bufs[b], gsem[b])

    def body(k, carry):
        for b in range(PIPE):
            j = k * PIPE + b
            pltpu.make_async_copy(t_hbm.at[src_v.at[j]], bufs[b],
                                  gsem[b]).wait()
            pltpu.sync_copy(bufs[b], acc_sp.at[dst_v.at[j]], add=True)

            @pl.when(j + PIPE < NWIN)
            def _():
                pltpu.async_copy(t_hbm.at[src_v.at[j + PIPE]], bufs[b],
                                 gsem[b])
        return carry

    lax.fori_loop(0, NBLK, body, 0)
    plsc.subcore_barrier()
    pltpu.sync_copy(acc_sp.at[pl.ds(s * SLICE, SLICE)],
                    out_hbm.at[c, pl.ds(s * SLICE, SLICE)])


def _tc_proj1(x, W1, dout_col):
    def body(x_ref, w_ref, dg_ref, o_ref):
        scale = lax.rsqrt(jnp.maximum(dg_ref[...], 1.0))
        t1 = jnp.dot(x_ref[...], w_ref[...],
                     preferred_element_type=jnp.float32) * scale
        o_ref[...] = jnp.concatenate(
            [t1, jnp.zeros((NPAD - N, HID), jnp.float32)], axis=0)

    return pl.pallas_call(
        body,
        out_shape=jax.ShapeDtypeStruct((NPAD, HID), jnp.float32),
    )(x, W1, dout_col)


def _tc_mid(acc, din_col, dout_col, b1_row, W2):
    def body(a_ref, din_ref, dout_ref, b_ref, w_ref, o_ref):
        din_s = lax.rsqrt(jnp.maximum(din_ref[...], 1.0))
        dout_s = lax.rsqrt(jnp.maximum(dout_ref[...], 1.0))
        agg = a_ref[0, :N] + a_ref[1, :N]
        h = jnp.maximum(agg * din_s + b_ref[...], 0.0)
        t2 = jnp.dot(h, w_ref[...],
                     preferred_element_type=jnp.float32) * dout_s
        o_ref[...] = jnp.concatenate(
            [t2, jnp.zeros((NPAD - N, HID), jnp.float32)], axis=0)

    return pl.pallas_call(
        body,
        out_shape=jax.ShapeDtypeStruct((NPAD, HID), jnp.float32),
    )(acc, din_col, dout_col, b1_row, W2)


def _tc_head(acc, din_col, b2_row, Wmu, bmu_row, Wlv, blv_row):
    def body(a_ref, din_ref, b_ref, wmu_ref, bmu_ref, wlv_ref, blv_ref, o_ref):
        din_s = lax.rsqrt(jnp.maximum(din_ref[...], 1.0))
        agg = a_ref[0, :N] + a_ref[1, :N]
        h = jnp.maximum(agg * din_s + b_ref[...], 0.0)
        hg = jnp.sum(h, axis=0, keepdims=True) * (1.0 / N)
        mu = jnp.dot(hg, wmu_ref[...], preferred_element_type=jnp.float32)
        lv = jnp.dot(hg, wlv_ref[...], preferred_element_type=jnp.float32)
        o_ref[...] = jnp.concatenate([mu + bmu_ref[...], lv + blv_ref[...]],
                                     axis=0)

    return pl.pallas_call(
        body,
        out_shape=jax.ShapeDtypeStruct((2, LAT), jnp.float32),
    )(acc, din_col, b2_row, Wmu, bmu_row, Wlv, blv_row)


def kernel(x, edge_index, edge_feat, W1, b1, W2, b2, We, be, Wmu, bmu, Wlv, blv):
    pad_idx = N + (jnp.arange(EPW_PAD - EPW, dtype=jnp.int32) % (NPAD - N))
    pad_blk = jnp.broadcast_to(pad_idx, (NW, EPW_PAD - EPW))
    src3 = jnp.concatenate(
        [edge_index[0].reshape(NW, EPW), pad_blk], axis=1).reshape(NW, NWIN, WIN)
    dst3 = jnp.concatenate(
        [edge_index[1].reshape(NW, EPW), pad_blk], axis=1).reshape(NW, NWIN, WIN)
    zeros2d = jnp.zeros((NPAD, HID), jnp.float32)

    deg_out2, deg_in2 = _deg_kernel(src3, dst3)
    dout_col = (deg_out2[0, :N] + deg_out2[1, :N]).reshape(N, 1)
    din_col = (deg_in2[0, :N] + deg_in2[1, :N]).reshape(N, 1)

    t1 = _tc_proj1(x, W1, dout_col)
    acc1 = _scatter_kernel(t1, src3, dst3, zeros2d)
    t2 = _tc_mid(acc1, din_col, dout_col, b1.reshape(1, HID), W2)
    acc2 = _scatter_kernel(t2, src3, dst3, zeros2d)
    out = _tc_head(acc2, din_col, b2.reshape(1, HID),
                   Wmu, bmu.reshape(1, LAT), Wlv, blv.reshape(1, LAT))
    return out[0:1], out[1:2]
